# baseline (speedup 1.0000x reference)
import functools

import jax
import jax.numpy as jnp
from jax import lax
from jax.experimental import pallas as pl
from jax.experimental.pallas import tpu as pltpu
from jax.experimental.pallas import tpu_sc as plsc

N = 10000
E = 320000
D = 128
B = 16

R = 10000
NB = N // R
NCORE = 2
NSUB = 16
NT = NCORE * NSUB
EPT = E // NT
CH = 125
CHUNKS = EPT // CH
N_PAD = 10240
RPT = N_PAD // NSUB


def _vn_add_body(x_ref, vn_ref, o_ref):
    o_ref[...] = x_ref[...] + vn_ref[...]


def _sc_agg_body(hin_hbm, eidx_hbm, zero_hbm, out_hbm,
                 src_v, dst_v, rows_v, agg_sh, sem_is, sem_id, sem_g, sem_s):
    cid = lax.axis_index("c")
    sid = lax.axis_index("s")
    wid = cid * NSUB + sid
    pltpu.sync_copy(zero_hbm, rows_v.at[0, pl.ds(0, 120)])
    for piece in range(5):
        pltpu.sync_copy(rows_v.at[0, pl.ds(0, 120)],
                        agg_sh.at[pl.ds(sid * RPT + 120 * piece, 120)])
    pltpu.sync_copy(rows_v.at[0, pl.ds(0, 40)],
                    agg_sh.at[pl.ds(sid * RPT + 600, 40)])
    pltpu.sync_copy(eidx_hbm.at[0, wid, 0], src_v.at[0])
    pltpu.sync_copy(eidx_hbm.at[1, wid, 0], dst_v.at[0])
    pltpu.async_copy(eidx_hbm.at[0, wid, 1], src_v.at[1], sem_is.at[1])
    pltpu.async_copy(eidx_hbm.at[1, wid, 1], dst_v.at[1], sem_id.at[1])
    pltpu.async_copy(hin_hbm.at[src_v.at[0]], rows_v.at[0], sem_g.at[0])
    plsc.subcore_barrier()

    def body(j, carry):
        slot = lax.rem(j, 2)
        nslot = lax.rem(j + 1, 2)
        d_slot = lax.rem(j, 3)

        @pl.when(j < CHUNKS - 1)
        def _():
            @pl.when(j >= 1)
            def _():
                pltpu.make_async_copy(rows_v.at[nslot],
                                      agg_sh.at[dst_v.at[lax.rem(j - 1, 3)]],
                                      sem_s.at[nslot]).wait()

            pltpu.make_async_copy(eidx_hbm.at[0, wid, j + 1],
                                  src_v.at[nslot], sem_is.at[nslot]).wait()
            pltpu.async_copy(hin_hbm.at[src_v.at[nslot]], rows_v.at[nslot],
                             sem_g.at[nslot])

        pltpu.make_async_copy(hin_hbm.at[src_v.at[slot]], rows_v.at[slot],
                              sem_g.at[slot]).wait()

        @pl.when(j >= 1)
        def _():
            pltpu.make_async_copy(eidx_hbm.at[1, wid, j], dst_v.at[d_slot],
                                  sem_id.at[d_slot]).wait()

        pltpu.async_copy(rows_v.at[slot], agg_sh.at[dst_v.at[d_slot]],
                         sem_s.at[slot], add=True)

        @pl.when(j < CHUNKS - 2)
        def _():
            pltpu.async_copy(eidx_hbm.at[0, wid, j + 2], src_v.at[slot],
                             sem_is.at[slot])
            pltpu.async_copy(eidx_hbm.at[1, wid, j + 2],
                             dst_v.at[lax.rem(j + 2, 3)],
                             sem_id.at[lax.rem(j + 2, 3)])

        return carry

    lax.fori_loop(0, CHUNKS, body, 0)
    pltpu.make_async_copy(rows_v.at[0], agg_sh.at[dst_v.at[0]],
                          sem_s.at[0]).wait()
    pltpu.make_async_copy(rows_v.at[1], agg_sh.at[dst_v.at[1]],
                          sem_s.at[1]).wait()
    plsc.subcore_barrier()
    pltpu.sync_copy(agg_sh.at[pl.ds(sid * RPT, RPT)],
                    out_hbm.at[cid, pl.ds(sid * RPT, RPT)])


@functools.cache
def _sc_agg():
    return pl.kernel(
        _sc_agg_body,
        out_type=jax.ShapeDtypeStruct((NCORE, N_PAD, D), jnp.float32),
        mesh=plsc.VectorSubcoreMesh(core_axis_name="c", subcore_axis_name="s",
                                    num_cores=NCORE, num_subcores=NSUB),
        scratch_types=[
            pltpu.VMEM((2, CH), jnp.int32),
            pltpu.VMEM((3, CH), jnp.int32),
            pltpu.VMEM((2, CH, D), jnp.float32),
            pltpu.VMEM_SHARED((N_PAD, D), jnp.float32),
            pltpu.SemaphoreType.DMA((2,)),
            pltpu.SemaphoreType.DMA((3,)),
            pltpu.SemaphoreType.DMA((2,)),
            pltpu.SemaphoreType.DMA((2,)),
        ],
    )


def _tc_body(p_ref, w_ref, b_ref, bid_ref, vn_ref, wlin_ref,
             blin_ref, gam_ref, bet_ref, w1_ref, b1_ref, g1_ref, be1_ref,
             w2_ref, b2_ref, g2_ref, be2_ref,
             h_ref, t_ref,
             zs_ref, cs_ref, cq_ref, seg_ref, cnt_ref, scale_ref, shift_ref):
    g = pl.program_id(0)
    p = g // NB
    i = lax.rem(g, NB)

    @pl.when(p == 0)
    def _():
        z = (jnp.dot(p_ref[0], w_ref[...], preferred_element_type=jnp.float32)
             + jnp.dot(p_ref[1], w_ref[...], preferred_element_type=jnp.float32)
             + b_ref[...])
        zs_ref[i] = z
        bid = bid_ref[0]
        ohT = (lax.broadcasted_iota(jnp.int32, (B, R), 0) == bid)
        ohT = ohT.astype(jnp.float32)
        seg = jnp.dot(ohT, z, preferred_element_type=jnp.float32)
        cnt = jnp.sum(ohT, axis=1, keepdims=True) * jnp.ones((1, D),
                                                             jnp.float32)
        cs = jnp.sum(z, axis=0, keepdims=True)
        cq = jnp.sum(z * z, axis=0, keepdims=True)

        @pl.when(i == 0)
        def _():
            cs_ref[...] = cs
            cq_ref[...] = cq
            seg_ref[...] = seg
            cnt_ref[...] = cnt

        @pl.when(i != 0)
        def _():
            cs_ref[...] += cs
            cq_ref[...] += cq
            seg_ref[...] += seg
            cnt_ref[...] += cnt

    @pl.when(jnp.logical_and(p == 1, i == 0))
    def _():
        m = cs_ref[...] * (1.0 / N)
        v = cq_ref[...] * (1.0 / N) - m * m
        inv = lax.rsqrt(v + 1e-5)
        scale = inv * gam_ref[...]
        shift = bet_ref[...] - m * scale
        scale_ref[...] = scale
        shift_ref[...] = shift
        cnt = cnt_ref[:, 0:1]
        pooled = seg_ref[...] * scale + cnt * shift
        vn_lin = jnp.dot(vn_ref[...], wlin_ref[...],
                         preferred_element_type=jnp.float32) + blin_ref[...]
        pooled = pooled + vn_lin
        t = jnp.dot(pooled, w1_ref[...],
                    preferred_element_type=jnp.float32) + b1_ref[...]
        mt = jnp.mean(t, axis=0, keepdims=True)
        vt = jnp.mean((t - mt) * (t - mt), axis=0, keepdims=True)
        t = (t - mt) * lax.rsqrt(vt + 1e-5) * g1_ref[...] + be1_ref[...]
        t = jnp.maximum(t, 0.0)
        t = jnp.dot(t, w2_ref[...],
                    preferred_element_type=jnp.float32) + b2_ref[...]
        mt = jnp.mean(t, axis=0, keepdims=True)
        vt = jnp.mean((t - mt) * (t - mt), axis=0, keepdims=True)
        t = (t - mt) * lax.rsqrt(vt + 1e-5) * g2_ref[...] + be2_ref[...]
        t_ref[...] = jnp.maximum(t, 0.0)

    @pl.when(p == 1)
    def _():
        h_ref[...] = zs_ref[i] * scale_ref[...] + shift_ref[...]


def kernel(x, edge_index, batch_id, vn_table, Wconv, bconv, bn_gamma, bn_beta,
           Wlin, blin, W1, b1, g1, be1, W2, b2, g2, be2):
    eidx = edge_index.astype(jnp.int32).reshape(2, NT, CHUNKS, CH)
    bid = batch_id.astype(jnp.int32).reshape(NB, 1, R)
    zero = jnp.zeros((120, D), jnp.float32)
    row = lambda v: v.reshape(1, D)

    hin = pl.pallas_call(
        _vn_add_body,
        grid=(NB,),
        in_specs=[pl.BlockSpec((R, D), lambda i: (i, 0)),
                  pl.BlockSpec((1, D), lambda i: (0, 0))],
        out_specs=pl.BlockSpec((R, D), lambda i: (i, 0)),
        out_shape=jax.ShapeDtypeStruct((N, D), jnp.float32),
    )(x, vn_table)

    partials = _sc_agg()(hin, eidx, zero)

    h, t = pl.pallas_call(
        _tc_body,
        grid=(2 * NB,),
        in_specs=[pl.BlockSpec((NCORE, R, D),
                               lambda g: (0, jnp.minimum(g, NB - 1), 0)),
                  pl.BlockSpec((D, D), lambda g: (0, 0)),
                  pl.BlockSpec((1, D), lambda g: (0, 0)),
                  pl.BlockSpec((1, 1, R),
                               lambda g: (jnp.minimum(g, NB - 1), 0, 0)),
                  pl.BlockSpec((1, D), lambda g: (0, 0)),
                  pl.BlockSpec((D, D), lambda g: (0, 0)),
                  pl.BlockSpec((1, D), lambda g: (0, 0)),
                  pl.BlockSpec((1, D), lambda g: (0, 0)),
                  pl.BlockSpec((1, D), lambda g: (0, 0)),
                  pl.BlockSpec((D, D), lambda g: (0, 0)),
                  pl.BlockSpec((1, D), lambda g: (0, 0)),
                  pl.BlockSpec((1, D), lambda g: (0, 0)),
                  pl.BlockSpec((1, D), lambda g: (0, 0)),
                  pl.BlockSpec((D, D), lambda g: (0, 0)),
                  pl.BlockSpec((1, D), lambda g: (0, 0)),
                  pl.BlockSpec((1, D), lambda g: (0, 0)),
                  pl.BlockSpec((1, D), lambda g: (0, 0))],
        out_specs=[pl.BlockSpec((R, D),
                                lambda g: (jnp.maximum(g - NB, 0), 0)),
                   pl.BlockSpec((B, D), lambda g: (0, 0))],
        out_shape=[jax.ShapeDtypeStruct((N, D), jnp.float32),
                   jax.ShapeDtypeStruct((B, D), jnp.float32)],
        scratch_shapes=[pltpu.VMEM((NB, R, D), jnp.float32),
                        pltpu.VMEM((1, D), jnp.float32),
                        pltpu.VMEM((1, D), jnp.float32),
                        pltpu.VMEM((B, D), jnp.float32),
                        pltpu.VMEM((B, D), jnp.float32),
                        pltpu.VMEM((1, D), jnp.float32),
                        pltpu.VMEM((1, D), jnp.float32)],
    )(partials, Wconv, row(bconv), bid, vn_table, Wlin, row(blin),
      row(bn_gamma), row(bn_beta), W1, row(b1), row(g1), row(be1), W2,
      row(b2), row(g2), row(be2))
    return h, t


# speedup vs baseline: 1.0074x; 1.0074x over previous
import functools

import jax
import jax.numpy as jnp
from jax import lax
from jax.experimental import pallas as pl
from jax.experimental.pallas import tpu as pltpu
from jax.experimental.pallas import tpu_sc as plsc

N = 10000
E = 320000
D = 128
B = 16

R = 5000
NB = N // R
NCORE = 2
NSUB = 16
NT = NCORE * NSUB
EPT = E // NT
CH = 125
CHUNKS = EPT // CH
N_PAD = 10240
RPT = N_PAD // NSUB


def _vn_add_body(x_ref, vn_ref, o_ref):
    o_ref[...] = x_ref[...] + vn_ref[...]


def _sc_agg_body(hin_hbm, eidx_hbm, zero_hbm, out_hbm,
                 src_v, dst_v, rows_v, agg_sh, sem_is, sem_id, sem_g, sem_s):
    cid = lax.axis_index("c")
    sid = lax.axis_index("s")
    wid = cid * NSUB + sid
    pltpu.sync_copy(zero_hbm, rows_v.at[0, pl.ds(0, 120)])
    for piece in range(5):
        pltpu.sync_copy(rows_v.at[0, pl.ds(0, 120)],
                        agg_sh.at[pl.ds(sid * RPT + 120 * piece, 120)])
    pltpu.sync_copy(rows_v.at[0, pl.ds(0, 40)],
                    agg_sh.at[pl.ds(sid * RPT + 600, 40)])
    pltpu.sync_copy(eidx_hbm.at[0, wid, 0], src_v.at[0])
    pltpu.sync_copy(eidx_hbm.at[1, wid, 0], dst_v.at[0])
    pltpu.async_copy(eidx_hbm.at[0, wid, 1], src_v.at[1], sem_is.at[1])
    pltpu.async_copy(eidx_hbm.at[1, wid, 1], dst_v.at[1], sem_id.at[1])
    pltpu.async_copy(hin_hbm.at[src_v.at[0]], rows_v.at[0], sem_g.at[0])
    plsc.subcore_barrier()

    def body(j, carry):
        slot = lax.rem(j, 2)
        nslot = lax.rem(j + 1, 2)
        d_slot = lax.rem(j, 3)

        @pl.when(j < CHUNKS - 1)
        def _():
            @pl.when(j >= 1)
            def _():
                pltpu.make_async_copy(rows_v.at[nslot],
                                      agg_sh.at[dst_v.at[lax.rem(j - 1, 3)]],
                                      sem_s.at[nslot]).wait()

            pltpu.make_async_copy(eidx_hbm.at[0, wid, j + 1],
                                  src_v.at[nslot], sem_is.at[nslot]).wait()
            pltpu.async_copy(hin_hbm.at[src_v.at[nslot]], rows_v.at[nslot],
                             sem_g.at[nslot])

        pltpu.make_async_copy(hin_hbm.at[src_v.at[slot]], rows_v.at[slot],
                              sem_g.at[slot]).wait()

        @pl.when(j >= 1)
        def _():
            pltpu.make_async_copy(eidx_hbm.at[1, wid, j], dst_v.at[d_slot],
                                  sem_id.at[d_slot]).wait()

        pltpu.async_copy(rows_v.at[slot], agg_sh.at[dst_v.at[d_slot]],
                         sem_s.at[slot], add=True)

        @pl.when(j < CHUNKS - 2)
        def _():
            pltpu.async_copy(eidx_hbm.at[0, wid, j + 2], src_v.at[slot],
                             sem_is.at[slot])
            pltpu.async_copy(eidx_hbm.at[1, wid, j + 2],
                             dst_v.at[lax.rem(j + 2, 3)],
                             sem_id.at[lax.rem(j + 2, 3)])

        return carry

    lax.fori_loop(0, CHUNKS, body, 0)
    pltpu.make_async_copy(rows_v.at[0], agg_sh.at[dst_v.at[0]],
                          sem_s.at[0]).wait()
    pltpu.make_async_copy(rows_v.at[1], agg_sh.at[dst_v.at[1]],
                          sem_s.at[1]).wait()
    plsc.subcore_barrier()
    pltpu.sync_copy(agg_sh.at[pl.ds(sid * RPT, RPT)],
                    out_hbm.at[cid, pl.ds(sid * RPT, RPT)])


@functools.cache
def _sc_agg():
    return pl.kernel(
        _sc_agg_body,
        out_type=jax.ShapeDtypeStruct((NCORE, N_PAD, D), jnp.float32),
        mesh=plsc.VectorSubcoreMesh(core_axis_name="c", subcore_axis_name="s",
                                    num_cores=NCORE, num_subcores=NSUB),
        scratch_types=[
            pltpu.VMEM((2, CH), jnp.int32),
            pltpu.VMEM((3, CH), jnp.int32),
            pltpu.VMEM((2, CH, D), jnp.float32),
            pltpu.VMEM_SHARED((N_PAD, D), jnp.float32),
            pltpu.SemaphoreType.DMA((2,)),
            pltpu.SemaphoreType.DMA((3,)),
            pltpu.SemaphoreType.DMA((2,)),
            pltpu.SemaphoreType.DMA((2,)),
        ],
    )


def _tc_body(p_ref, w_ref, b_ref, bid_ref, vn_ref, wlin_ref,
             blin_ref, gam_ref, bet_ref, w1_ref, b1_ref, g1_ref, be1_ref,
             w2_ref, b2_ref, g2_ref, be2_ref,
             h_ref, t_ref,
             zs_ref, cs_ref, cq_ref, seg_ref, cnt_ref, scale_ref, shift_ref):
    g = pl.program_id(0)
    p = g // NB
    i = lax.rem(g, NB)

    @pl.when(p == 0)
    def _():
        z = (jnp.dot(p_ref[0], w_ref[...], preferred_element_type=jnp.float32)
             + jnp.dot(p_ref[1], w_ref[...], preferred_element_type=jnp.float32)
             + b_ref[...])
        zs_ref[i] = z
        bid = bid_ref[0]
        ohT = (lax.broadcasted_iota(jnp.int32, (B, R), 0) == bid)
        ohT = ohT.astype(jnp.float32)
        seg = jnp.dot(ohT, z, preferred_element_type=jnp.float32)
        cnt = jnp.sum(ohT, axis=1, keepdims=True) * jnp.ones((1, D),
                                                             jnp.float32)
        cs = jnp.sum(z, axis=0, keepdims=True)
        cq = jnp.sum(z * z, axis=0, keepdims=True)

        @pl.when(i == 0)
        def _():
            cs_ref[...] = cs
            cq_ref[...] = cq
            seg_ref[...] = seg
            cnt_ref[...] = cnt

        @pl.when(i != 0)
        def _():
            cs_ref[...] += cs
            cq_ref[...] += cq
            seg_ref[...] += seg
            cnt_ref[...] += cnt

    @pl.when(jnp.logical_and(p == 1, i == 0))
    def _():
        m = cs_ref[...] * (1.0 / N)
        v = cq_ref[...] * (1.0 / N) - m * m
        inv = lax.rsqrt(v + 1e-5)
        scale = inv * gam_ref[...]
        shift = bet_ref[...] - m * scale
        scale_ref[...] = scale
        shift_ref[...] = shift
        cnt = cnt_ref[:, 0:1]
        pooled = seg_ref[...] * scale + cnt * shift
        vn_lin = jnp.dot(vn_ref[...], wlin_ref[...],
                         preferred_element_type=jnp.float32) + blin_ref[...]
        pooled = pooled + vn_lin
        t = jnp.dot(pooled, w1_ref[...],
                    preferred_element_type=jnp.float32) + b1_ref[...]
        mt = jnp.mean(t, axis=0, keepdims=True)
        vt = jnp.mean((t - mt) * (t - mt), axis=0, keepdims=True)
        t = (t - mt) * lax.rsqrt(vt + 1e-5) * g1_ref[...] + be1_ref[...]
        t = jnp.maximum(t, 0.0)
        t = jnp.dot(t, w2_ref[...],
                    preferred_element_type=jnp.float32) + b2_ref[...]
        mt = jnp.mean(t, axis=0, keepdims=True)
        vt = jnp.mean((t - mt) * (t - mt), axis=0, keepdims=True)
        t = (t - mt) * lax.rsqrt(vt + 1e-5) * g2_ref[...] + be2_ref[...]
        t_ref[...] = jnp.maximum(t, 0.0)

    @pl.when(p == 1)
    def _():
        h_ref[...] = zs_ref[i] * scale_ref[...] + shift_ref[...]


def kernel(x, edge_index, batch_id, vn_table, Wconv, bconv, bn_gamma, bn_beta,
           Wlin, blin, W1, b1, g1, be1, W2, b2, g2, be2):
    eidx = edge_index.astype(jnp.int32).reshape(2, NT, CHUNKS, CH)
    bid = batch_id.astype(jnp.int32).reshape(NB, 1, R)
    zero = jnp.zeros((120, D), jnp.float32)
    row = lambda v: v.reshape(1, D)

    hin = pl.pallas_call(
        _vn_add_body,
        grid=(NB,),
        in_specs=[pl.BlockSpec((R, D), lambda i: (i, 0)),
                  pl.BlockSpec((1, D), lambda i: (0, 0))],
        out_specs=pl.BlockSpec((R, D), lambda i: (i, 0)),
        out_shape=jax.ShapeDtypeStruct((N, D), jnp.float32),
    )(x, vn_table)

    partials = _sc_agg()(hin, eidx, zero)

    h, t = pl.pallas_call(
        _tc_body,
        grid=(2 * NB,),
        in_specs=[pl.BlockSpec((NCORE, R, D),
                               lambda g: (0, jnp.minimum(g, NB - 1), 0)),
                  pl.BlockSpec((D, D), lambda g: (0, 0)),
                  pl.BlockSpec((1, D), lambda g: (0, 0)),
                  pl.BlockSpec((1, 1, R),
                               lambda g: (jnp.minimum(g, NB - 1), 0, 0)),
                  pl.BlockSpec((1, D), lambda g: (0, 0)),
                  pl.BlockSpec((D, D), lambda g: (0, 0)),
                  pl.BlockSpec((1, D), lambda g: (0, 0)),
                  pl.BlockSpec((1, D), lambda g: (0, 0)),
                  pl.BlockSpec((1, D), lambda g: (0, 0)),
                  pl.BlockSpec((D, D), lambda g: (0, 0)),
                  pl.BlockSpec((1, D), lambda g: (0, 0)),
                  pl.BlockSpec((1, D), lambda g: (0, 0)),
                  pl.BlockSpec((1, D), lambda g: (0, 0)),
                  pl.BlockSpec((D, D), lambda g: (0, 0)),
                  pl.BlockSpec((1, D), lambda g: (0, 0)),
                  pl.BlockSpec((1, D), lambda g: (0, 0)),
                  pl.BlockSpec((1, D), lambda g: (0, 0))],
        out_specs=[pl.BlockSpec((R, D),
                                lambda g: (jnp.maximum(g - NB, 0), 0)),
                   pl.BlockSpec((B, D), lambda g: (0, 0))],
        out_shape=[jax.ShapeDtypeStruct((N, D), jnp.float32),
                   jax.ShapeDtypeStruct((B, D), jnp.float32)],
        scratch_shapes=[pltpu.VMEM((NB, R, D), jnp.float32),
                        pltpu.VMEM((1, D), jnp.float32),
                        pltpu.VMEM((1, D), jnp.float32),
                        pltpu.VMEM((B, D), jnp.float32),
                        pltpu.VMEM((B, D), jnp.float32),
                        pltpu.VMEM((1, D), jnp.float32),
                        pltpu.VMEM((1, D), jnp.float32)],
    )(partials, Wconv, row(bconv), bid, vn_table, Wlin, row(blin),
      row(bn_gamma), row(bn_beta), W1, row(b1), row(g1), row(be1), W2,
      row(b2), row(g2), row(be2))
    return h, t
